# baseline (speedup 1.0000x reference)
import functools

import jax
import jax.numpy as jnp
from jax import lax
from jax.experimental import pallas as pl
from jax.experimental.pallas import tpu as pltpu
from jax.experimental.pallas import tpu_sc as plsc

N = 4096
L = 16
NC = 2
NS = 16
NW = NC * NS
PADN = N + L
LISTN = PADN + L
NUM_IDS = 256
IDS_PER_W = NUM_IDS // NW


def _sc_body(x_hbm, out_hbm,
             x_v, list_v, idg_v, list2_v, sg_v, qg_v, segend_v, part_v):
    cid = lax.axis_index("c")
    sid = lax.axis_index("s")
    wid = sid * NC + cid

    pltpu.sync_copy(x_hbm, x_v)

    lo = wid * IDS_PER_W
    hi = lo + IDS_PER_W
    iota = lax.iota(jnp.int32, L)
    zero_i = jnp.zeros((L,), jnp.int32)
    one_i = jnp.ones((L,), jnp.int32)
    trash = jnp.full((L,), LISTN - 1, jnp.int32)

    def _take(v, idx):
        return lax.gather(
            v, idx[:, None],
            lax.GatherDimensionNumbers(
                offset_dims=(), collapsed_slice_dims=(0,),
                start_index_map=(0,)),
            (1,), mode=lax.GatherScatterMode.PROMISE_IN_BOUNDS)

    def _prefix(mi):
        pref = mi
        for s in (1, 2, 4, 8):
            shifted = _take(pref, jnp.maximum(iota - s, 0))
            pref = pref + jnp.where(iota >= s, shifted, zero_i)
        return pref

    def compact(k, pos_vec):
        res = pos_vec
        for u in range(2):
            i0 = (2 * k + u) * L
            v = plsc.bitcast(x_v[pl.ds(i0, L)], jnp.int32)
            m = (v >= lo) & (v < hi)
            mi = jnp.where(m, one_i, zero_i)
            pref = _prefix(mi)
            dest = jnp.where(m, res + pref - 1, trash)
            plsc.store_scatter(list_v, [dest], iota + i0)
            plsc.store_scatter(idg_v, [dest], v)
            res = res + plsc.all_reduce_population_count(m)
        return res

    pos_vec = lax.fori_loop(0, N // L // 2, compact, zero_i)
    kk = pos_vec[0]
    list_v[pl.ds(kk, L)] = jnp.full((L,), N, jnp.int32)
    idg_v[pl.ds(kk, L)] = jnp.full((L,), -1, jnp.int32)
    idg_v[pl.ds(kk + L, L)] = jnp.full((L,), -1, jnp.int32)

    gtrips = lax.div(kk + (L - 1), L)
    gtrips2 = lax.div(gtrips + 1, 2)

    offs = [jnp.int32(0)]
    pos2 = zero_i
    for pi in range(IDS_PER_W):
        pid = lo + pi

        def compact2(k, pos_vec2, pid=pid):
            res = pos_vec2
            for u in range(2):
                i0 = (2 * k + u) * L
                m = idg_v[pl.ds(i0, L)] == pid
                mi = jnp.where(m, one_i, zero_i)
                pref = _prefix(mi)
                dest = jnp.where(m, res + pref - 1, trash)
                plsc.store_scatter(list2_v, [dest], list_v[pl.ds(i0, L)])
                res = res + plsc.all_reduce_population_count(m)
            return res

        pos2 = lax.fori_loop(0, gtrips2, compact2, pos2)
        offs.append(pos2[0])
    list2_v[pl.ds(kk, L)] = jnp.full((L,), N, jnp.int32)

    for pi in range(IDS_PER_W):
        o = offs[pi]
        end = offs[pi + 1]
        end_splat = jnp.zeros((L,), jnp.int32) + end
        nst = lax.div(end - o + (2 * L - 1), L)

        def segstore(b, _, o=o, end_splat=end_splat):
            segend_v[pl.ds(o + b * L, L)] = end_splat
            return 0

        lax.fori_loop(0, nst, segstore, 0)

    def gather_sq(b, _):
        i0 = b * L
        il = list2_v[pl.ds(i0, L)]
        sg = plsc.load_gather(x_v, [il + PADN])
        sg_v[pl.ds(i0, L)] = sg
        qg_v[pl.ds(i0, L)] = sg - plsc.load_gather(x_v, [il + 2 * PADN])
        return 0

    lax.fori_loop(0, gtrips, gather_sq, 0)
    sg_v[pl.ds(kk, L)] = jnp.zeros((L,), jnp.float32)
    qg_v[pl.ds(kk, L)] = jnp.zeros((L,), jnp.float32)

    zero = jnp.zeros((L,), jnp.float32)

    acc, cnt = zero, zero
    ttrips = lax.div(kk + 1, 2)

    def tstep0(u, carry):
        acc, cnt = carry
        for w in range(2):
            t = 2 * u + w
            a_s = _take(sg_v[pl.ds(t, L)], zero_i)
            a_q = _take(qg_v[pl.ds(t, L)], zero_i)
            end_sp = _take(segend_v[pl.ds(t, L)], zero_i)
            base = t + 1
            sd = a_s - sg_v[pl.ds(base, L)]
            dq = a_q - qg_v[pl.ds(base, L)]
            sgn = jnp.sign(sd)
            hinge = jnp.maximum(sgn * dq, 0.0)
            lane_ok = (base + iota) < end_sp
            acc = acc + jnp.where(lane_ok, hinge, zero)
            cnt = cnt + jnp.where(lane_ok, jnp.abs(sgn), zero)
        return acc, cnt

    acc, cnt = lax.fori_loop(0, ttrips, tstep0, (acc, cnt))

    for pi in range(IDS_PER_W):
        o = offs[pi]
        end = offs[pi + 1]
        kp = end - o
        nblocks = lax.div(kp + (L - 2), L)

        def block(j, carry, o=o, end=end):
            nt = end - 1 - j * L - o
            trips = lax.div(nt + 1, 2)

            def tstep(u, carry):
                acc, cnt = carry
                for w in range(2):
                    t = o + 2 * u + w
                    a_s = _take(sg_v[pl.ds(t, L)], zero_i)
                    a_q = _take(qg_v[pl.ds(t, L)], zero_i)
                    base = t + 1 + j * L
                    sd = a_s - sg_v[pl.ds(base, L)]
                    dq = a_q - qg_v[pl.ds(base, L)]
                    sgn = jnp.sign(sd)
                    hinge = jnp.maximum(sgn * dq, 0.0)
                    lane_ok = (base + iota) < end
                    acc = acc + jnp.where(lane_ok, hinge, zero)
                    cnt = cnt + jnp.where(lane_ok, jnp.abs(sgn), zero)
                return acc, cnt

            return lax.fori_loop(0, trips, tstep, carry)

        acc, cnt = lax.fori_loop(1, nblocks, block, (acc, cnt))

    part_v[pl.ds(0, L)] = acc
    part_v[pl.ds(L, L)] = cnt
    pltpu.sync_copy(part_v, out_hbm.at[pl.ds(wid * 2 * L, 2 * L)])


@jax.jit
def _pairwise_loss_sc(ids, s, p):
    idpad = jnp.full((L,), -1, jnp.int32)
    zpad = jnp.zeros((L,), jnp.float32)
    x = jnp.concatenate([
        lax.bitcast_convert_type(ids, jnp.float32),
        lax.bitcast_convert_type(idpad, jnp.float32),
        s, zpad, p, zpad,
    ])
    mesh = plsc.VectorSubcoreMesh(core_axis_name="c", subcore_axis_name="s")
    run = pl.kernel(
        _sc_body,
        mesh=mesh,
        compiler_params=pltpu.CompilerParams(needs_layout_passes=False),
        out_type=jax.ShapeDtypeStruct((NW * 2 * L,), jnp.float32),
        scratch_types=[
            pltpu.VMEM((3 * PADN,), jnp.float32),
            pltpu.VMEM((LISTN,), jnp.int32),
            pltpu.VMEM((LISTN,), jnp.int32),
            pltpu.VMEM((LISTN,), jnp.int32),
            pltpu.VMEM((PADN,), jnp.float32),
            pltpu.VMEM((PADN,), jnp.float32),
            pltpu.VMEM((LISTN,), jnp.int32),
            pltpu.VMEM((2 * L,), jnp.float32),
        ],
    )
    parts = run(x).reshape(NW, 2, L)
    loss_sum = jnp.sum(parts[:, 0, :])
    valid_cnt = jnp.sum(parts[:, 1, :])
    return loss_sum / jnp.maximum(valid_cnt, 1.0)


def kernel(reg_output, stress_scores, patient_ids):
    pred = jnp.squeeze(reg_output, -1)
    return _pairwise_loss_sc(patient_ids, stress_scores, pred)


# speedup vs baseline: 1.0190x; 1.0190x over previous
import jax
import jax.numpy as jnp
from jax import lax
from jax.experimental import pallas as pl
from jax.experimental.pallas import tpu as pltpu
from jax.experimental.pallas import tpu_sc as plsc

N = 4096
L = 16
NC = 2
NS = 16
NW = NC * NS
PADN = N + L
LISTN = PADN + L
NUM_IDS = 256
IDS_PER_W = NUM_IDS // NW


def _sc_body(x_hbm, out_hbm,
             x_v, list_v, idg_v, list2_v, sg_v, qg_v, part_v):
    cid = lax.axis_index("c")
    sid = lax.axis_index("s")
    wid = sid * NC + cid

    pltpu.sync_copy(x_hbm, x_v)

    lo = wid * IDS_PER_W
    hi = lo + IDS_PER_W
    iota = lax.iota(jnp.int32, L)
    zero_i = jnp.zeros((L,), jnp.int32)
    one_i = jnp.ones((L,), jnp.int32)
    trash = jnp.full((L,), LISTN - 1, jnp.int32)

    def _take(v, idx):
        return lax.gather(
            v, idx[:, None],
            lax.GatherDimensionNumbers(
                offset_dims=(), collapsed_slice_dims=(0,),
                start_index_map=(0,)),
            (1,), mode=lax.GatherScatterMode.PROMISE_IN_BOUNDS)

    def _prefix(mi):
        pref = mi
        for s in (1, 2, 4, 8):
            shifted = _take(pref, jnp.maximum(iota - s, 0))
            pref = pref + jnp.where(iota >= s, shifted, zero_i)
        return pref

    def compact(k, pos_vec):
        res = pos_vec
        for u in range(2):
            i0 = (2 * k + u) * L
            v = plsc.bitcast(x_v[pl.ds(i0, L)], jnp.int32)
            m = (v >= lo) & (v < hi)
            mi = jnp.where(m, one_i, zero_i)
            pref = _prefix(mi)
            dest = jnp.where(m, res + pref - 1, trash)
            plsc.store_scatter(list_v, [dest], iota + i0)
            plsc.store_scatter(idg_v, [dest], v)
            res = res + plsc.all_reduce_population_count(m)
        return res

    pos_vec = lax.fori_loop(0, N // L // 2, compact, zero_i)
    kk = pos_vec[0]
    list_v[pl.ds(kk, L)] = jnp.full((L,), N, jnp.int32)
    idg_v[pl.ds(kk, L)] = jnp.full((L,), -1, jnp.int32)
    idg_v[pl.ds(kk + L, L)] = jnp.full((L,), -1, jnp.int32)

    gtrips = lax.div(kk + (L - 1), L)
    gtrips2 = lax.div(gtrips + 1, 2)

    offs = [jnp.int32(0)]
    pos2 = zero_i
    for pi in range(IDS_PER_W):
        pid = lo + pi

        def compact2(k, pos_vec2, pid=pid):
            res = pos_vec2
            for u in range(2):
                i0 = (2 * k + u) * L
                m = idg_v[pl.ds(i0, L)] == pid
                mi = jnp.where(m, one_i, zero_i)
                pref = _prefix(mi)
                dest = jnp.where(m, res + pref - 1, trash)
                plsc.store_scatter(list2_v, [dest], list_v[pl.ds(i0, L)])
                res = res + plsc.all_reduce_population_count(m)
            return res

        pos2 = lax.fori_loop(0, gtrips2, compact2, pos2)
        offs.append(pos2[0])
    list2_v[pl.ds(kk, L)] = jnp.full((L,), N, jnp.int32)

    def gather_sq(b, _):
        i0 = b * L
        il = list2_v[pl.ds(i0, L)]
        sg = plsc.load_gather(x_v, [il + PADN])
        sg_v[pl.ds(i0, L)] = sg
        qg_v[pl.ds(i0, L)] = sg - plsc.load_gather(x_v, [il + 2 * PADN])
        return 0

    lax.fori_loop(0, gtrips, gather_sq, 0)
    sg_v[pl.ds(kk, L)] = jnp.zeros((L,), jnp.float32)
    qg_v[pl.ds(kk, L)] = jnp.zeros((L,), jnp.float32)

    zero = jnp.zeros((L,), jnp.float32)

    acc, cnt = zero, zero
    for pi in range(IDS_PER_W):
        o = offs[pi]
        end = offs[pi + 1]
        kp = end - o
        nblocks = lax.div(kp + (L - 2), L)

        def block(j, carry, o=o, end=end):
            nt = end - 1 - j * L - o
            trips = lax.div(nt + 1, 2)

            def tstep(u, carry):
                acc, cnt = carry
                for w in range(2):
                    t = o + 2 * u + w
                    a_s = _take(sg_v[pl.ds(t, L)], zero_i)
                    a_q = _take(qg_v[pl.ds(t, L)], zero_i)
                    base = t + 1 + j * L
                    sd = a_s - sg_v[pl.ds(base, L)]
                    dq = a_q - qg_v[pl.ds(base, L)]
                    sgn = jnp.sign(sd)
                    hinge = jnp.maximum(sgn * dq, 0.0)
                    lane_ok = (base + iota) < end
                    acc = acc + jnp.where(lane_ok, hinge, zero)
                    cnt = cnt + jnp.where(lane_ok, jnp.abs(sgn), zero)
                return acc, cnt

            return lax.fori_loop(0, trips, tstep, carry)

        acc, cnt = lax.fori_loop(0, nblocks, block, (acc, cnt))

    part_v[pl.ds(0, L)] = acc
    part_v[pl.ds(L, L)] = cnt
    pltpu.sync_copy(part_v, out_hbm.at[pl.ds(wid * 2 * L, 2 * L)])


@jax.jit
def _pairwise_loss_sc(ids, s, p):
    idpad = jnp.full((L,), -1, jnp.int32)
    zpad = jnp.zeros((L,), jnp.float32)
    x = jnp.concatenate([
        lax.bitcast_convert_type(ids, jnp.float32),
        lax.bitcast_convert_type(idpad, jnp.float32),
        s, zpad, p, zpad,
    ])
    mesh = plsc.VectorSubcoreMesh(core_axis_name="c", subcore_axis_name="s")
    run = pl.kernel(
        _sc_body,
        mesh=mesh,
        compiler_params=pltpu.CompilerParams(needs_layout_passes=False),
        out_type=jax.ShapeDtypeStruct((NW * 2 * L,), jnp.float32),
        scratch_types=[
            pltpu.VMEM((3 * PADN,), jnp.float32),
            pltpu.VMEM((LISTN,), jnp.int32),
            pltpu.VMEM((LISTN,), jnp.int32),
            pltpu.VMEM((LISTN,), jnp.int32),
            pltpu.VMEM((PADN,), jnp.float32),
            pltpu.VMEM((PADN,), jnp.float32),
            pltpu.VMEM((2 * L,), jnp.float32),
        ],
    )
    parts = run(x).reshape(NW, 2, L)
    loss_sum = jnp.sum(parts[:, 0, :])
    valid_cnt = jnp.sum(parts[:, 1, :])
    return loss_sum / jnp.maximum(valid_cnt, 1.0)


def kernel(reg_output, stress_scores, patient_ids):
    pred = jnp.squeeze(reg_output, -1)
    return _pairwise_loss_sc(patient_ids, stress_scores, pred)
